# baseline (speedup 1.0000x reference)
import functools

import numpy as np
import jax
import jax.numpy as jnp
from jax import lax
from jax.experimental import pallas as pl
from jax.experimental.pallas import tpu as pltpu
from jax.experimental.pallas import tpu_sc as plsc

L = 16
CHUNK = 32
DMA_ROWS = 128


def _make_sc_gather_sum(n_points, n_per_batch, D, K):
    info = plsc.get_sparse_core_info()
    nc, ns = info.num_cores, info.num_subcores
    nw = nc * ns
    ppw = n_points // nw
    n_chunks = ppw // CHUNK
    rpc = CHUNK * K
    n_dma = rpc // DMA_ROWS
    assert ppw * nw == n_points and n_chunks * CHUNK == ppw
    assert n_dma * DMA_ROWS == rpc
    assert n_per_batch % ppw == 0

    na = D // L
    mesh = plsc.VectorSubcoreMesh(core_axis_name="c", subcore_axis_name="s")

    @functools.partial(
        pl.kernel,
        mesh=mesh,
        out_type=jax.ShapeDtypeStruct((n_points * D,), jnp.float32),
        scratch_types=[
            pltpu.VMEM((ppw * K,), jnp.int32),
            pltpu.VMEM((2 * rpc, D), jnp.float32),
            pltpu.VMEM((2, CHUNK * D), jnp.float32),
            pltpu.SemaphoreType.DMA,
            pltpu.SemaphoreType.DMA,
        ],
        compiler_params=pltpu.CompilerParams(
            needs_layout_passes=False, use_tc_tiling_on_sc=False),
    )
    def sc_kernel(adj_hbm, x_hbm, out_hbm,
                  adj_v, rows_v, out_v, sem, sem_out):
        wid = lax.axis_index("s") * nc + lax.axis_index("c")
        tile_base = wid * ppw
        batch_idx = tile_base // n_per_batch

        pltpu.sync_copy(adj_hbm.at[pl.ds(tile_base * K, ppw * K)], adj_v)
        xb_hbm = x_hbm.at[batch_idx]
        lanes = lax.iota(jnp.int32, L)
        pos_t = [lanes * K + t for t in range(K)]
        zrow = jnp.zeros((L,), jnp.int32)

        def fetch_chunk(g, par):
            for i in range(n_dma):
                pltpu.async_copy(
                    xb_hbm.at[adj_v.at[pl.ds(g * rpc + i * DMA_ROWS, DMA_ROWS)]],
                    rows_v.at[pl.ds(par * rpc + i * DMA_ROWS, DMA_ROWS)],
                    sem)

        fetch_chunk(0, 0)

        def chunk_body(g, carry):
            par = lax.rem(g, 2)
            rbase0 = par * rpc
            pltpu.make_async_copy(
                xb_hbm.at[pl.ds(0, rpc)],
                rows_v.at[pl.ds(rbase0, rpc)],
                sem).wait()

            @pl.when(g + 1 < n_chunks)
            def _():
                fetch_chunk(g + 1, 1 - par)

            @pl.when(g >= 2)
            def _():
                pltpu.make_async_copy(
                    out_v.at[par], out_hbm.at[pl.ds(0, CHUNK * D)],
                    sem_out).wait()

            def pt_body(p, c2):
                fbase = (rbase0 + p * K) * D
                for a in range(na):
                    base = fbase + (5 * D) * a
                    gs = [plsc.load_gather(rows_v, [zrow, pos_t[t] + base])
                          for t in range(K)]
                    while len(gs) > 1:
                        gs = [gs[i] + gs[i + 1] for i in range(0, len(gs) - 1, 2)] \
                            + ([gs[-1]] if len(gs) % 2 else [])
                    out_v[par, pl.ds(p * D + L * a, L)] = gs[0]
                return c2

            lax.fori_loop(0, CHUNK, pt_body, 0)
            pltpu.async_copy(
                out_v.at[par],
                out_hbm.at[pl.ds((tile_base + g * CHUNK) * D, CHUNK * D)],
                sem_out)
            return carry

        lax.fori_loop(0, n_chunks, chunk_body, 0)
        for _ in range(2):
            pltpu.make_async_copy(
                out_v.at[0], out_hbm.at[pl.ds(0, CHUNK * D)], sem_out).wait()

    return sc_kernel


def _tc_linear(msum, xf, wa, wc, bias8):
    n, d = msum.shape
    out_c = wa.shape[1]
    bm = 1024
    grid = n // bm

    def body(m_ref, x_ref, a_ref, c_ref, b_ref, o_ref):
        o_ref[...] = (
            jnp.dot(m_ref[...], a_ref[...],
                    preferred_element_type=jnp.float32)
            + jnp.dot(x_ref[...], c_ref[...],
                      preferred_element_type=jnp.float32)
            + b_ref[0:1, :])

    return pl.pallas_call(
        body,
        grid=(grid,),
        in_specs=[
            pl.BlockSpec((bm, d), lambda i: (i, 0)),
            pl.BlockSpec((bm, d), lambda i: (i, 0)),
            pl.BlockSpec((d, out_c), lambda i: (0, 0)),
            pl.BlockSpec((d, out_c), lambda i: (0, 0)),
            pl.BlockSpec((8, out_c), lambda i: (0, 0)),
        ],
        out_specs=pl.BlockSpec((bm, out_c), lambda i: (i, 0)),
        out_shape=jax.ShapeDtypeStruct((n, out_c), jnp.float32),
    )(msum, xf, wa, wc, bias8)


def kernel(x, adj, W, b):
    B, N, D = x.shape
    K = adj.shape[-1]
    out_c = W.shape[0]

    xf = x.reshape(B * N, D)
    msum_flat = _make_sc_gather_sum(B * N, N, D, K)(
        adj.reshape(B * N * K), x)
    msum = msum_flat.reshape(B * N, D)

    w1 = W[:, :D]
    w2 = W[:, D:]
    wa = (w1.T / K).astype(jnp.float32)
    wc = (w2 - w1).T.astype(jnp.float32)
    bias8 = jnp.broadcast_to(b.reshape(1, out_c), (8, out_c))

    out2d = _tc_linear(msum, xf, wa, wc, bias8)
    return out2d.reshape(B, N, out_c)


# speedup vs baseline: 1.0133x; 1.0133x over previous
import functools

import numpy as np
import jax
import jax.numpy as jnp
from jax import lax
from jax.experimental import pallas as pl
from jax.experimental.pallas import tpu as pltpu
from jax.experimental.pallas import tpu_sc as plsc

L = 16
CHUNK = 32
DMA_ROWS = 128


def _make_sc_gather_sum(n_points, n_per_batch, D, K):
    info = plsc.get_sparse_core_info()
    nc, ns = info.num_cores, info.num_subcores
    nw = nc * ns
    ppw = n_points // nw
    n_chunks = ppw // CHUNK
    rpc = CHUNK * K
    n_dma = rpc // DMA_ROWS
    assert ppw * nw == n_points and n_chunks * CHUNK == ppw
    assert n_dma * DMA_ROWS == rpc
    assert n_per_batch % ppw == 0

    na = D // L
    dd = 2 * D
    mesh = plsc.VectorSubcoreMesh(core_axis_name="c", subcore_axis_name="s")

    @functools.partial(
        pl.kernel,
        mesh=mesh,
        out_type=jax.ShapeDtypeStruct((n_points * dd,), jnp.float32),
        scratch_types=[
            pltpu.VMEM((ppw * K,), jnp.int32),
            pltpu.VMEM((2 * rpc, D), jnp.float32),
            pltpu.VMEM((2 * CHUNK, D), jnp.float32),
            pltpu.VMEM((2, CHUNK * dd), jnp.float32),
            pltpu.SemaphoreType.DMA,
            pltpu.SemaphoreType.DMA,
        ],
        compiler_params=pltpu.CompilerParams(
            needs_layout_passes=False, use_tc_tiling_on_sc=False),
    )
    def sc_kernel(adj_hbm, x_hbm, out_hbm,
                  adj_v, rows_v, xrow_v, out_v, sem, sem_out):
        wid = lax.axis_index("s") * nc + lax.axis_index("c")
        tile_base = wid * ppw
        batch_idx = tile_base // n_per_batch

        pltpu.sync_copy(adj_hbm.at[pl.ds(tile_base * K, ppw * K)], adj_v)
        xb_hbm = x_hbm.at[batch_idx]
        lanes = lax.iota(jnp.int32, L)
        pos_t = [lanes * K + t for t in range(K)]
        zrow = jnp.zeros((L,), jnp.int32)

        def fetch_chunk(g, par):
            for i in range(n_dma):
                pltpu.async_copy(
                    xb_hbm.at[adj_v.at[pl.ds(g * rpc + i * DMA_ROWS, DMA_ROWS)]],
                    rows_v.at[pl.ds(par * rpc + i * DMA_ROWS, DMA_ROWS)],
                    sem)
            pltpu.async_copy(
                xb_hbm.at[pl.ds(tile_base - batch_idx * n_per_batch
                                + g * CHUNK, CHUNK)],
                xrow_v.at[pl.ds(par * CHUNK, CHUNK)],
                sem)

        fetch_chunk(0, 0)

        def chunk_body(g, carry):
            par = lax.rem(g, 2)
            rbase0 = par * rpc
            pltpu.make_async_copy(
                xb_hbm.at[pl.ds(0, rpc)],
                rows_v.at[pl.ds(rbase0, rpc)],
                sem).wait()
            pltpu.make_async_copy(
                xb_hbm.at[pl.ds(0, CHUNK)],
                xrow_v.at[pl.ds(par * CHUNK, CHUNK)],
                sem).wait()

            @pl.when(g + 1 < n_chunks)
            def _():
                fetch_chunk(g + 1, 1 - par)

            @pl.when(g >= 2)
            def _():
                pltpu.make_async_copy(
                    out_v.at[par], out_hbm.at[pl.ds(0, CHUNK * D)],
                    sem_out).wait()

            def pt_body(p, c2):
                fbase = (rbase0 + p * K) * D
                for a in range(na):
                    base = fbase + (5 * D) * a
                    gs = [plsc.load_gather(rows_v, [zrow, pos_t[t] + base])
                          for t in range(K)]
                    while len(gs) > 1:
                        gs = [gs[i] + gs[i + 1] for i in range(0, len(gs) - 1, 2)] \
                            + ([gs[-1]] if len(gs) % 2 else [])
                    out_v[par, pl.ds(p * dd + L * a, L)] = gs[0]
                    out_v[par, pl.ds(p * dd + D + L * a, L)] = \
                        xrow_v[par * CHUNK + p, pl.ds(L * a, L)]
                return c2

            lax.fori_loop(0, CHUNK, pt_body, 0)
            pltpu.async_copy(
                out_v.at[par],
                out_hbm.at[pl.ds((tile_base + g * CHUNK) * dd, CHUNK * dd)],
                sem_out)
            return carry

        lax.fori_loop(0, n_chunks, chunk_body, 0)
        for _ in range(2):
            pltpu.make_async_copy(
                out_v.at[0], out_hbm.at[pl.ds(0, CHUNK * dd)], sem_out).wait()

    return sc_kernel


def _tc_linear(mpacked, wcat, bias8):
    n, dd = mpacked.shape
    out_c = wcat.shape[1]
    bm = 1024
    grid = n // bm

    def body(m_ref, w_ref, b_ref, o_ref):
        o_ref[...] = (
            jnp.dot(m_ref[...], w_ref[...],
                    preferred_element_type=jnp.float32)
            + b_ref[0:1, :])

    return pl.pallas_call(
        body,
        grid=(grid,),
        in_specs=[
            pl.BlockSpec((bm, dd), lambda i: (i, 0)),
            pl.BlockSpec((dd, out_c), lambda i: (0, 0)),
            pl.BlockSpec((8, out_c), lambda i: (0, 0)),
        ],
        out_specs=pl.BlockSpec((bm, out_c), lambda i: (i, 0)),
        out_shape=jax.ShapeDtypeStruct((n, out_c), jnp.float32),
    )(mpacked, wcat, bias8)


def kernel(x, adj, W, b):
    B, N, D = x.shape
    K = adj.shape[-1]
    out_c = W.shape[0]

    mp_flat = _make_sc_gather_sum(B * N, N, D, K)(
        adj.reshape(B * N * K), x)
    mpacked = mp_flat.reshape(B * N, 2 * D)

    w1 = W[:, :D]
    w2 = W[:, D:]
    wcat = jnp.concatenate([w1.T / K, (w2 - w1).T], axis=0).astype(jnp.float32)
    bias8 = jnp.broadcast_to(b.reshape(1, out_c), (8, out_c))

    out2d = _tc_linear(mpacked, wcat, bias8)
    return out2d.reshape(B, N, out_c)


# speedup vs baseline: 1.1152x; 1.1005x over previous
import functools

import numpy as np
import jax
import jax.numpy as jnp
from jax import lax
from jax.experimental import pallas as pl
from jax.experimental.pallas import tpu as pltpu
from jax.experimental.pallas import tpu_sc as plsc

L = 16
CHUNK = 32
DMA_ROWS = 128


def _make_sc_gather_sum(n_points, n_per_batch, D, K):
    info = plsc.get_sparse_core_info()
    nc, ns = info.num_cores, info.num_subcores
    nw = nc * ns
    ppw = n_points // nw
    n_chunks = ppw // CHUNK
    rpc = CHUNK * K
    n_dma = rpc // DMA_ROWS
    assert ppw * nw == n_points and n_chunks * CHUNK == ppw
    assert n_dma * DMA_ROWS == rpc
    assert n_per_batch % ppw == 0

    na = D // L
    dd = 2 * D
    mesh = plsc.VectorSubcoreMesh(core_axis_name="c", subcore_axis_name="s")

    @functools.partial(
        pl.kernel,
        mesh=mesh,
        out_type=jax.ShapeDtypeStruct((n_points * dd,), jnp.float32),
        scratch_types=[
            pltpu.VMEM((K, ppw), jnp.int32),
            pltpu.VMEM((2 * rpc, D), jnp.float32),
            pltpu.VMEM((2 * CHUNK, D), jnp.float32),
            pltpu.VMEM((2, CHUNK * dd), jnp.float32),
            pltpu.SemaphoreType.DMA,
            pltpu.SemaphoreType.DMA,
        ],
        compiler_params=pltpu.CompilerParams(
            needs_layout_passes=False, use_tc_tiling_on_sc=False),
    )
    def sc_kernel(adjt_hbm, x_hbm, out_hbm,
                  adj_v, rows_v, xrow_v, out_v, sem, sem_out):
        wid = lax.axis_index("s") * nc + lax.axis_index("c")
        tile_base = wid * ppw
        batch_idx = tile_base // n_per_batch
        n0 = tile_base - batch_idx * n_per_batch

        for k in range(K):
            pltpu.async_copy(adjt_hbm.at[k, batch_idx, pl.ds(n0, ppw)],
                             adj_v.at[k], sem)
        xb_hbm = x_hbm.at[batch_idx]
        for k in range(K):
            pltpu.make_async_copy(adjt_hbm.at[k, batch_idx, pl.ds(n0, ppw)],
                                  adj_v.at[k], sem).wait()
        lanes = lax.iota(jnp.int32, L)
        pos_t = []
        for t in range(K):
            q = lanes * K + t
            pos_t.append((q // D) * (CHUNK * D) + lax.rem(q, D))
        zrow = jnp.zeros((L,), jnp.int32)

        def fetch_chunk(g, par):
            for k in range(K):
                pltpu.async_copy(
                    xb_hbm.at[adj_v.at[k, pl.ds(g * CHUNK, CHUNK)]],
                    rows_v.at[pl.ds((par * K + k) * CHUNK, CHUNK)],
                    sem)
            pltpu.async_copy(
                xb_hbm.at[pl.ds(n0 + g * CHUNK, CHUNK)],
                xrow_v.at[pl.ds(par * CHUNK, CHUNK)],
                sem)

        fetch_chunk(0, 0)

        def chunk_body(g, carry):
            par = lax.rem(g, 2)
            rbase0 = par * rpc
            pltpu.make_async_copy(
                xb_hbm.at[pl.ds(0, rpc)],
                rows_v.at[pl.ds(rbase0, rpc)],
                sem).wait()
            pltpu.make_async_copy(
                xb_hbm.at[pl.ds(0, CHUNK)],
                xrow_v.at[pl.ds(par * CHUNK, CHUNK)],
                sem).wait()

            @pl.when(g + 1 < n_chunks)
            def _():
                fetch_chunk(g + 1, 1 - par)

            @pl.when(g >= 2)
            def _():
                pltpu.make_async_copy(
                    out_v.at[par], out_hbm.at[pl.ds(0, CHUNK * D)],
                    sem_out).wait()

            def pt_body(p, c2):
                fbase = rbase0 * D + p * D
                for a in range(na):
                    base = fbase + (5 * CHUNK * D) * a
                    gs = [plsc.load_gather(rows_v, [zrow, pos_t[t] + base])
                          for t in range(K)]
                    while len(gs) > 1:
                        gs = [gs[i] + gs[i + 1] for i in range(0, len(gs) - 1, 2)] \
                            + ([gs[-1]] if len(gs) % 2 else [])
                    out_v[par, pl.ds(p * dd + L * a, L)] = gs[0]
                    out_v[par, pl.ds(p * dd + D + L * a, L)] = \
                        xrow_v[par * CHUNK + p, pl.ds(L * a, L)]
                return c2

            lax.fori_loop(0, CHUNK, pt_body, 0)
            pltpu.async_copy(
                out_v.at[par],
                out_hbm.at[pl.ds((tile_base + g * CHUNK) * dd, CHUNK * dd)],
                sem_out)
            return carry

        lax.fori_loop(0, n_chunks, chunk_body, 0)
        for _ in range(2):
            pltpu.make_async_copy(
                out_v.at[0], out_hbm.at[pl.ds(0, CHUNK * dd)], sem_out).wait()

    return sc_kernel


def _tc_linear(mpacked, wcat, bias8):
    n, dd = mpacked.shape
    out_c = wcat.shape[1]
    bm = 1024
    grid = n // bm

    def body(m_ref, w_ref, b_ref, o_ref):
        o_ref[...] = (
            jnp.dot(m_ref[...], w_ref[...],
                    preferred_element_type=jnp.float32)
            + b_ref[0:1, :])

    return pl.pallas_call(
        body,
        grid=(grid,),
        in_specs=[
            pl.BlockSpec((bm, dd), lambda i: (i, 0)),
            pl.BlockSpec((dd, out_c), lambda i: (0, 0)),
            pl.BlockSpec((8, out_c), lambda i: (0, 0)),
        ],
        out_specs=pl.BlockSpec((bm, out_c), lambda i: (i, 0)),
        out_shape=jax.ShapeDtypeStruct((n, out_c), jnp.float32),
    )(mpacked, wcat, bias8)


def kernel(x, adj, W, b):
    B, N, D = x.shape
    K = adj.shape[-1]
    out_c = W.shape[0]

    mp_flat = _make_sc_gather_sum(B * N, N, D, K)(
        jnp.transpose(adj, (2, 0, 1)), x)
    mpacked = mp_flat.reshape(B * N, 2 * D)

    w1 = W[:, :D]
    w2 = W[:, D:]
    wcat = jnp.concatenate([w1.T / K, (w2 - w1).T], axis=0).astype(jnp.float32)
    bias8 = jnp.broadcast_to(b.reshape(1, out_c), (8, out_c))

    out2d = _tc_linear(mpacked, wcat, bias8)
    return out2d.reshape(B, N, out_c)
